# baseline (speedup 1.0000x reference)
import functools

import jax
import jax.numpy as jnp
from jax import lax
from jax.experimental import pallas as pl
from jax.experimental.pallas import tpu as pltpu
from jax.experimental.pallas import tpu_sc as plsc

DIM = 128
ITERS = 4
NC = 2
NS = 16
CH = 128


def _mlp4_body(nblk, embs, W1, b1, W2, b2, out):
    j = pl.program_id(0)
    del nblk
    x = embs[0]
    h = jnp.maximum(jnp.dot(x, W1[0], preferred_element_type=jnp.float32) + b1[0], 0.0)
    out[...] = jnp.dot(h, W2[0], preferred_element_type=jnp.float32) + b2[0]


def _mlp4(embs, W1s, b1s, W2s, b2s, blk):
    n = embs.shape[1]
    nblk = n // blk
    return pl.pallas_call(
        functools.partial(_mlp4_body, nblk),
        grid=(4, nblk),
        in_specs=[
            pl.BlockSpec((1, blk, DIM), lambda j, i: (1 - j // 2, i, 0)),
            pl.BlockSpec((1, DIM, DIM), lambda j, i: (j, 0, 0)),
            pl.BlockSpec((1, 1, DIM), lambda j, i: (j, 0, 0)),
            pl.BlockSpec((1, DIM, DIM), lambda j, i: (j, 0, 0)),
            pl.BlockSpec((1, 1, DIM), lambda j, i: (j, 0, 0)),
        ],
        out_specs=pl.BlockSpec((blk, DIM), lambda j, i: (j * nblk + i, 0)),
        out_shape=jax.ShapeDtypeStruct((4 * n, DIM), jnp.float32),
    )(embs, W1s, b1s, W2s, b2s)


def _gru_body(ap, an, embs, WihT, WhhT, bih, bhh, out):
    x_p = ap[...]
    x_n = an[...]
    h = embs[0]
    gi = (jnp.dot(x_p, WihT[0, :DIM], preferred_element_type=jnp.float32)
          + jnp.dot(x_n, WihT[0, DIM:], preferred_element_type=jnp.float32)
          + bih[0])
    gh = jnp.dot(h, WhhT[0], preferred_element_type=jnp.float32) + bhh[0]
    r = jax.nn.sigmoid(gi[:, :DIM] + gh[:, :DIM])
    z = jax.nn.sigmoid(gi[:, DIM:2 * DIM] + gh[:, DIM:2 * DIM])
    nn = jnp.tanh(gi[:, 2 * DIM:] + r * gh[:, 2 * DIM:])
    out[0] = (1.0 - z) * nn + z * h


def _gru2(msgs, embs, WihT_s, WhhT_s, bih_s, bhh_s, blk):
    n = embs.shape[1]
    nblk = n // blk
    return pl.pallas_call(
        _gru_body,
        grid=(2, nblk),
        in_specs=[
            pl.BlockSpec((blk, DIM), lambda s, i: (2 * s * nblk + i, 0)),
            pl.BlockSpec((blk, DIM), lambda s, i: ((2 * s + 1) * nblk + i, 0)),
            pl.BlockSpec((1, blk, DIM), lambda s, i: (s, i, 0)),
            pl.BlockSpec((1, 2 * DIM, 3 * DIM), lambda s, i: (s, 0, 0)),
            pl.BlockSpec((1, DIM, 3 * DIM), lambda s, i: (s, 0, 0)),
            pl.BlockSpec((1, 1, 3 * DIM), lambda s, i: (s, 0, 0)),
            pl.BlockSpec((1, 1, 3 * DIM), lambda s, i: (s, 0, 0)),
        ],
        out_specs=pl.BlockSpec((1, blk, DIM), lambda s, i: (s, i, 0)),
        out_shape=jax.ShapeDtypeStruct((2, n, DIM), jnp.float32),
    )(msgs, msgs, embs, WihT_s, WhhT_s, bih_s, bhh_s)


SUPER = 2


def _make_sc_seg(n, nck):
    rows_per = (n // NS) // 8 * 8
    rem = n - NS * rows_per
    zb = rows_per // 6
    assert zb * 6 == rows_per and zb % 8 == 0 and rem % 8 == 0 and rem <= zb
    assert nck % SUPER == 0
    nsk = nck // SUPER
    mesh = plsc.VectorSubcoreMesh(core_axis_name="c", subcore_axis_name="s",
                                  num_cores=NC, num_subcores=NS)

    @functools.partial(
        pl.kernel,
        out_type=jax.ShapeDtypeStruct((4 * n, DIM), jnp.float32),
        mesh=mesh,
        scratch_types=[
            pltpu.VMEM_SHARED((n + 8, DIM), jnp.float32),
            pltpu.VMEM((SUPER * CH,), jnp.int32),
            pltpu.VMEM((SUPER * CH,), jnp.int32),
            pltpu.VMEM((SUPER * CH, DIM), jnp.float32),
            pltpu.VMEM((zb, DIM), jnp.float32),
            pltpu.SemaphoreType.DMA,
        ],
    )
    def sc_seg(tables, srcs, dsts, zrows, out, acc, sbuf, dbuf, rows, zbuf,
               gsem):
        cid = lax.axis_index("c")
        sid = lax.axis_index("s")
        pltpu.sync_copy(zrows, zbuf)
        for step in range(2):
            j = cid * 2 + step
            for z in range(6):
                pltpu.sync_copy(zbuf, acc.at[pl.ds(sid * rows_per + z * zb, zb)])

            @pl.when(sid == NS - 1)
            def _():
                pltpu.sync_copy(zbuf.at[pl.ds(0, rem)],
                                acc.at[pl.ds(NS * rows_per, rem)])
            plsc.subcore_barrier()

            base = (j * NS + sid) * nck * CH

            def sc_body(t, carry):
                r = base + t * (SUPER * CH)
                pltpu.sync_copy(srcs.at[pl.ds(r, SUPER * CH)], sbuf)
                pltpu.sync_copy(dsts.at[pl.ds(r, SUPER * CH)], dbuf)
                pltpu.async_copy(tables.at[sbuf], rows, gsem).wait()
                pltpu.sync_copy(rows, acc.at[dbuf], add=True)
                return carry

            lax.fori_loop(0, nsk, sc_body, 0)
            plsc.subcore_barrier()
            pltpu.sync_copy(acc.at[pl.ds(sid * rows_per, rows_per)],
                            out.at[pl.ds(j * n + sid * rows_per, rows_per)])

            @pl.when(sid == NS - 1)
            def _():
                pltpu.sync_copy(acc.at[pl.ds(NS * rows_per, rem)],
                                out.at[pl.ds(j * n + NS * rows_per, rem)])

    return sc_seg


def kernel(v_size, c_size, v_edge_index, c_edge_index, p_edge_index,
           n_edge_index, v_emb, c_emb, params):
    del v_size, c_size
    n = v_emb.shape[0]
    ep = p_edge_index.shape[0]
    blk = 2000 if n % 2000 == 0 else n // 5

    vp = jnp.take(v_edge_index, p_edge_index).astype(jnp.int32)
    vn = jnp.take(v_edge_index, n_edge_index).astype(jnp.int32)
    cp = jnp.take(c_edge_index, p_edge_index).astype(jnp.int32)
    cn = jnp.take(c_edge_index, n_edge_index).astype(jnp.int32)
    nck = -(-(-(-ep // (NS * CH))) // SUPER) * SUPER
    epp = NS * nck * CH
    zpad = jnp.zeros((epp - ep,), jnp.int32)
    npad = jnp.full((epp - ep,), n, jnp.int32)
    srcs = jnp.concatenate([vp, zpad, vn + n, zpad, cp + 2 * n, zpad,
                            cn + 3 * n, zpad])
    dsts = jnp.concatenate([cp, npad, cn, npad, vp, npad, vn, npad])
    zrows = jnp.zeros(((n // NS) // 8 * 8 // 6, DIM), jnp.float32)

    W1s = jnp.stack([params[k][0] for k in
                     ('aggl_pos', 'aggl_neg', 'aggc_pos', 'aggc_neg')])
    b1s = jnp.stack([params[k][1] for k in
                     ('aggl_pos', 'aggl_neg', 'aggc_pos', 'aggc_neg')])[:, None, :]
    W2s = jnp.stack([params[k][2] for k in
                     ('aggl_pos', 'aggl_neg', 'aggc_pos', 'aggc_neg')])
    b2s = jnp.stack([params[k][3] for k in
                     ('aggl_pos', 'aggl_neg', 'aggc_pos', 'aggc_neg')])[:, None, :]
    WihT_s = jnp.stack([params['clause_upd'][0].T, params['variable_upd'][0].T])
    WhhT_s = jnp.stack([params['clause_upd'][1].T, params['variable_upd'][1].T])
    bih_s = jnp.stack([params['clause_upd'][2], params['variable_upd'][2]])[:, None, :]
    bhh_s = jnp.stack([params['clause_upd'][3], params['variable_upd'][3]])[:, None, :]

    sc_seg = _make_sc_seg(n, nck)

    embs = jnp.stack([c_emb, v_emb])
    v_list, c_list = [v_emb], [c_emb]
    for _ in range(ITERS):
        tables = _mlp4(embs, W1s, b1s, W2s, b2s, blk)
        msgs = sc_seg(tables, srcs, dsts, zrows)
        embs = _gru2(msgs, embs, WihT_s, WhhT_s, bih_s, bhh_s, blk)
        c_list.append(embs[0])
        v_list.append(embs[1])
    return (jnp.stack(v_list), jnp.stack(c_list))


# speedup vs baseline: 1.6909x; 1.6909x over previous
import functools

import jax
import jax.numpy as jnp
from jax import lax
from jax.experimental import pallas as pl
from jax.experimental.pallas import tpu as pltpu
from jax.experimental.pallas import tpu_sc as plsc

DIM = 128
ITERS = 4
NC = 2
NS = 16
CH = 128


def _mlp4_body(nblk, embs, W1, b1, W2, b2, out):
    j = pl.program_id(0)
    del nblk
    x = embs[0]
    h = jnp.maximum(jnp.dot(x, W1[0], preferred_element_type=jnp.float32) + b1[0], 0.0)
    out[...] = jnp.dot(h, W2[0], preferred_element_type=jnp.float32) + b2[0]


def _mlp4(embs, W1s, b1s, W2s, b2s, blk):
    n = embs.shape[1]
    nblk = n // blk
    return pl.pallas_call(
        functools.partial(_mlp4_body, nblk),
        grid=(4, nblk),
        in_specs=[
            pl.BlockSpec((1, blk, DIM), lambda j, i: (1 - j // 2, i, 0)),
            pl.BlockSpec((1, DIM, DIM), lambda j, i: (j, 0, 0)),
            pl.BlockSpec((1, 1, DIM), lambda j, i: (j, 0, 0)),
            pl.BlockSpec((1, DIM, DIM), lambda j, i: (j, 0, 0)),
            pl.BlockSpec((1, 1, DIM), lambda j, i: (j, 0, 0)),
        ],
        out_specs=pl.BlockSpec((blk, DIM), lambda j, i: (j * nblk + i, 0)),
        out_shape=jax.ShapeDtypeStruct((4 * n, DIM), jnp.float32),
    )(embs, W1s, b1s, W2s, b2s)


def _gru_body(ap, an, embs, WihT, WhhT, bih, bhh, out):
    x_p = ap[...]
    x_n = an[...]
    h = embs[0]
    gi = (jnp.dot(x_p, WihT[0, :DIM], preferred_element_type=jnp.float32)
          + jnp.dot(x_n, WihT[0, DIM:], preferred_element_type=jnp.float32)
          + bih[0])
    gh = jnp.dot(h, WhhT[0], preferred_element_type=jnp.float32) + bhh[0]
    r = jax.nn.sigmoid(gi[:, :DIM] + gh[:, :DIM])
    z = jax.nn.sigmoid(gi[:, DIM:2 * DIM] + gh[:, DIM:2 * DIM])
    nn = jnp.tanh(gi[:, 2 * DIM:] + r * gh[:, 2 * DIM:])
    out[0] = (1.0 - z) * nn + z * h


def _gru2(msgs, embs, WihT_s, WhhT_s, bih_s, bhh_s, blk):
    n = embs.shape[1]
    nblk = n // blk
    return pl.pallas_call(
        _gru_body,
        grid=(2, nblk),
        in_specs=[
            pl.BlockSpec((blk, DIM), lambda s, i: (2 * s * nblk + i, 0)),
            pl.BlockSpec((blk, DIM), lambda s, i: ((2 * s + 1) * nblk + i, 0)),
            pl.BlockSpec((1, blk, DIM), lambda s, i: (s, i, 0)),
            pl.BlockSpec((1, 2 * DIM, 3 * DIM), lambda s, i: (s, 0, 0)),
            pl.BlockSpec((1, DIM, 3 * DIM), lambda s, i: (s, 0, 0)),
            pl.BlockSpec((1, 1, 3 * DIM), lambda s, i: (s, 0, 0)),
            pl.BlockSpec((1, 1, 3 * DIM), lambda s, i: (s, 0, 0)),
        ],
        out_specs=pl.BlockSpec((1, blk, DIM), lambda s, i: (s, i, 0)),
        out_shape=jax.ShapeDtypeStruct((2, n, DIM), jnp.float32),
    )(msgs, msgs, embs, WihT_s, WhhT_s, bih_s, bhh_s)


def _make_sc_seg(n, ep):
    eptile = ep // NS
    nful, tail = divmod(eptile, CH)
    rows_per = (n // NS) // 8 * 8
    rem = n - NS * rows_per
    zb = rows_per // 3
    assert zb * 3 == rows_per and zb % 8 == 0 and rem % 8 == 0 and rem <= zb
    assert ep % NS == 0 and eptile % 8 == 0 and tail % 8 == 0
    mesh = plsc.VectorSubcoreMesh(core_axis_name="c", subcore_axis_name="s",
                                  num_cores=NC, num_subcores=NS)

    @functools.partial(
        pl.kernel,
        out_type=jax.ShapeDtypeStruct((4 * n, DIM), jnp.float32),
        mesh=mesh,
        scratch_types=[
            pltpu.VMEM_SHARED((n + 8, DIM), jnp.float32),
            pltpu.VMEM((CH,), jnp.int32),
            pltpu.VMEM((CH,), jnp.int32),
            pltpu.VMEM((CH, DIM), jnp.float32),
            pltpu.VMEM((max(tail, 8),), jnp.int32),
            pltpu.VMEM((max(tail, 8),), jnp.int32),
            pltpu.VMEM((max(tail, 8), DIM), jnp.float32),
            pltpu.VMEM((zb, DIM), jnp.float32),
            pltpu.SemaphoreType.DMA,
        ],
    )
    def sc_seg(tables, srcs, dsts, zrows, out, acc, sbuf, dbuf, rows,
               sbuf_t, dbuf_t, trows, zbuf, gsem):
        cid = lax.axis_index("c")
        sid = lax.axis_index("s")
        pltpu.sync_copy(zrows, zbuf)
        for step in range(2):
            j = cid * 2 + step
            for z in range(3):
                pltpu.sync_copy(zbuf, acc.at[pl.ds(sid * rows_per + z * zb, zb)])

            @pl.when(sid == NS - 1)
            def _():
                pltpu.sync_copy(zbuf.at[pl.ds(0, rem)],
                                acc.at[pl.ds(NS * rows_per, rem)])
            plsc.subcore_barrier()

            base = j * ep + sid * eptile

            def sc_body(k, carry):
                r = base + k * CH
                pltpu.sync_copy(srcs.at[pl.ds(r, CH)], sbuf)
                pltpu.sync_copy(dsts.at[pl.ds(r, CH)], dbuf)
                pltpu.async_copy(tables.at[sbuf], rows, gsem).wait()
                pltpu.sync_copy(rows, acc.at[dbuf], add=True)
                return carry

            lax.fori_loop(0, nful, sc_body, 0)
            if tail:
                r = base + nful * CH
                pltpu.sync_copy(srcs.at[pl.ds(r, tail)], sbuf_t)
                pltpu.sync_copy(dsts.at[pl.ds(r, tail)], dbuf_t)
                pltpu.async_copy(tables.at[sbuf_t], trows, gsem).wait()
                pltpu.sync_copy(trows, acc.at[dbuf_t], add=True)
            plsc.subcore_barrier()
            pltpu.sync_copy(acc.at[pl.ds(sid * rows_per, rows_per)],
                            out.at[pl.ds(j * n + sid * rows_per, rows_per)])

            @pl.when(sid == NS - 1)
            def _():
                pltpu.sync_copy(acc.at[pl.ds(NS * rows_per, rem)],
                                out.at[pl.ds(j * n + NS * rows_per, rem)])

    return sc_seg


def kernel(v_size, c_size, v_edge_index, c_edge_index, p_edge_index,
           n_edge_index, v_emb, c_emb, params):
    del v_size, c_size
    n = v_emb.shape[0]
    ep = p_edge_index.shape[0]
    blk = 2000 if n % 2000 == 0 else n // 5

    vp = jnp.take(v_edge_index, p_edge_index).astype(jnp.int32)
    vn = jnp.take(v_edge_index, n_edge_index).astype(jnp.int32)
    cp = jnp.take(c_edge_index, p_edge_index).astype(jnp.int32)
    cn = jnp.take(c_edge_index, n_edge_index).astype(jnp.int32)
    srcs = jnp.concatenate([vp, vn + n, cp + 2 * n, cn + 3 * n])
    dsts = jnp.concatenate([cp, cn, vp, vn])
    zrows = jnp.zeros(((n // NS) // 8 * 8 // 3, DIM), jnp.float32)

    W1s = jnp.stack([params[k][0] for k in
                     ('aggl_pos', 'aggl_neg', 'aggc_pos', 'aggc_neg')])
    b1s = jnp.stack([params[k][1] for k in
                     ('aggl_pos', 'aggl_neg', 'aggc_pos', 'aggc_neg')])[:, None, :]
    W2s = jnp.stack([params[k][2] for k in
                     ('aggl_pos', 'aggl_neg', 'aggc_pos', 'aggc_neg')])
    b2s = jnp.stack([params[k][3] for k in
                     ('aggl_pos', 'aggl_neg', 'aggc_pos', 'aggc_neg')])[:, None, :]
    WihT_s = jnp.stack([params['clause_upd'][0].T, params['variable_upd'][0].T])
    WhhT_s = jnp.stack([params['clause_upd'][1].T, params['variable_upd'][1].T])
    bih_s = jnp.stack([params['clause_upd'][2], params['variable_upd'][2]])[:, None, :]
    bhh_s = jnp.stack([params['clause_upd'][3], params['variable_upd'][3]])[:, None, :]

    sc_seg = _make_sc_seg(n, ep)

    embs = jnp.stack([c_emb, v_emb])
    v_list, c_list = [v_emb], [c_emb]
    for _ in range(ITERS):
        tables = _mlp4(embs, W1s, b1s, W2s, b2s, blk)
        msgs = sc_seg(tables, srcs, dsts, zrows)
        embs = _gru2(msgs, embs, WihT_s, WhhT_s, bih_s, bhh_s, blk)
        c_list.append(embs[0])
        v_list.append(embs[1])
    return (jnp.stack(v_list), jnp.stack(c_list))
